# baseline (speedup 1.0000x reference)
import functools

import jax
import jax.numpy as jnp
from jax import lax
from jax.experimental import pallas as pl
from jax.experimental.pallas import tpu as pltpu
from jax.experimental.pallas import tpu_sc as plsc

_NUM_CORES = 2
_NUM_SUBCORES = 16
_NW = _NUM_CORES * _NUM_SUBCORES
_L = 16

_D = 64
_DCH = _D // _L


def _vsqrt(x):
    xi = lax.bitcast_convert_type(x, jnp.int32)
    yi = jnp.int32(0x5F3759DF) - lax.shift_right_logical(xi, 1)
    y = lax.bitcast_convert_type(yi, jnp.float32)
    xh = x * jnp.float32(0.5)
    for _ in range(3):
        y = y * (jnp.float32(1.5) - xh * y * y)
    return x * y


def _transe_body(b_per_w, heads_hbm, rels_hbm, tails_hbm, ent_hbm, rel_hbm,
                 out_hbm, hidx_v, ridx_v, tidx_v, h_rows, r_rows, t_rows,
                 scores_v, sem):
    wid = lax.axis_index("s") * _NUM_CORES + lax.axis_index("c")
    base = wid * b_per_w

    pltpu.sync_copy(heads_hbm.at[pl.ds(base, b_per_w)], hidx_v)
    pltpu.sync_copy(rels_hbm.at[pl.ds(base, b_per_w)], ridx_v)
    pltpu.sync_copy(tails_hbm.at[pl.ds(base, b_per_w)], tidx_v)

    ch = pltpu.async_copy(ent_hbm.at[hidx_v], h_rows, sem)
    cr = pltpu.async_copy(rel_hbm.at[ridx_v], r_rows, sem)
    ct = pltpu.async_copy(ent_hbm.at[tidx_v], t_rows, sem)
    ch.wait()
    cr.wait()
    ct.wait()

    lanes = lax.iota(jnp.int32, _L)

    def group(g, carry):
        row_idx = g * _L + lanes
        acc = jnp.zeros((_L,), jnp.float32)
        for d in range(_D):
            col = jnp.full((_L,), d, jnp.int32)
            h = plsc.load_gather(h_rows, [row_idx, col])
            rl = plsc.load_gather(r_rows, [row_idx, col])
            t = plsc.load_gather(t_rows, [row_idx, col])
            diff = h + rl - t
            acc = acc + diff * diff
        scores_v[pl.ds(g * _L, _L)] = _vsqrt(acc)
        return carry

    lax.fori_loop(0, b_per_w // _L, group, 0)

    pltpu.sync_copy(scores_v, out_hbm.at[pl.ds(base, b_per_w)])


def kernel(heads, relations, tails, entity_emb, relation_emb):
    batch = heads.shape[0]
    b_per_w = batch // _NW
    mesh = plsc.VectorSubcoreMesh(core_axis_name="c", subcore_axis_name="s")

    k = pl.kernel(
        functools.partial(_transe_body, b_per_w),
        out_type=jax.ShapeDtypeStruct((batch,), jnp.float32),
        mesh=mesh,
        compiler_params=pltpu.CompilerParams(
            needs_layout_passes=False, use_tc_tiling_on_sc=False),
        scratch_types=[
            pltpu.VMEM((b_per_w,), jnp.int32),
            pltpu.VMEM((b_per_w,), jnp.int32),
            pltpu.VMEM((b_per_w,), jnp.int32),
            pltpu.VMEM((b_per_w, _D), jnp.float32),
            pltpu.VMEM((b_per_w, _D), jnp.float32),
            pltpu.VMEM((b_per_w, _D), jnp.float32),
            pltpu.VMEM((b_per_w,), jnp.float32),
            pltpu.SemaphoreType.DMA,
        ],
    )
    return k(heads.astype(jnp.int32), relations.astype(jnp.int32),
             tails.astype(jnp.int32), entity_emb, relation_emb)
